# Initial kernel scaffold; baseline (speedup 1.0000x reference)
import functools

import jax
import jax.numpy as jnp
from jax import lax
from jax.experimental import pallas as pl
from jax.experimental.pallas import tpu as pltpu
from jax.experimental.pallas import tpu_sc as plsc

_NC = 2
_NS = 16
_NW = _NC * _NS
_L = 16

_CH = 80
_CB = 2000


def _sc_degree(dst, ew, n):
    e = dst.shape[0]
    epw = e // _NW
    ncb = epw // _CB
    mesh = plsc.VectorSubcoreMesh(core_axis_name="c", subcore_axis_name="s")

    @functools.partial(
        pl.kernel,
        out_type=jax.ShapeDtypeStruct((_NW, n), jnp.float32),
        mesh=mesh,
        scratch_types=[
            pltpu.VMEM((_CB,), jnp.int32),
            pltpu.VMEM((_CB,), jnp.float32),
            pltpu.VMEM((n,), jnp.float32),
        ],
    )
    def deg_kernel(dst_hbm, ew_hbm, out_hbm, dbuf, wbuf, acc):
        wid = lax.axis_index("s") * _NC + lax.axis_index("c")
        zero16 = jnp.zeros((_L,), jnp.float32)

        @functools.partial(plsc.parallel_loop, 0, n // _L)
        def _zero(i):
            acc[pl.ds(i * _L, _L)] = zero16

        def chunk(c, carry):
            base = wid * epw + c * _CB
            pltpu.sync_copy(dst_hbm.at[pl.ds(base, _CB)], dbuf)
            pltpu.sync_copy(ew_hbm.at[pl.ds(base, _CB)], wbuf)

            def group(g, carry2):
                d16 = dbuf[pl.ds(g * _L, _L)]
                w16 = wbuf[pl.ds(g * _L, _L)]
                plsc.addupdate_scatter(acc, [d16], w16)
                return carry2

            return lax.fori_loop(0, _CB // _L, group, carry)

        lax.fori_loop(0, ncb, chunk, 0)
        pltpu.sync_copy(acc, out_hbm.at[wid])

    return deg_kernel(dst, ew)


def _sc_norm(src, dst, ew, dinv):
    e = src.shape[0]
    n = dinv.shape[0]
    epw = e // _NW
    ncb = epw // _CB
    mesh = plsc.VectorSubcoreMesh(core_axis_name="c", subcore_axis_name="s")

    @functools.partial(
        pl.kernel,
        out_type=jax.ShapeDtypeStruct((e,), jnp.float32),
        mesh=mesh,
        scratch_types=[
            pltpu.VMEM((n,), jnp.float32),
            pltpu.VMEM((_CB,), jnp.int32),
            pltpu.VMEM((_CB,), jnp.int32),
            pltpu.VMEM((_CB,), jnp.float32),
            pltpu.VMEM((_CB,), jnp.float32),
        ],
    )
    def norm_kernel(src_hbm, dst_hbm, ew_hbm, dinv_hbm, out_hbm,
                    dinv_v, sbuf, dbuf, wbuf, obuf):
        wid = lax.axis_index("s") * _NC + lax.axis_index("c")
        pltpu.sync_copy(dinv_hbm, dinv_v)

        def chunk(c, carry):
            base = wid * epw + c * _CB
            pltpu.sync_copy(src_hbm.at[pl.ds(base, _CB)], sbuf)
            pltpu.sync_copy(dst_hbm.at[pl.ds(base, _CB)], dbuf)
            pltpu.sync_copy(ew_hbm.at[pl.ds(base, _CB)], wbuf)

            @functools.partial(plsc.parallel_loop, 0, _CB // _L)
            def _group(g):
                s16 = sbuf[pl.ds(g * _L, _L)]
                d16 = dbuf[pl.ds(g * _L, _L)]
                w16 = wbuf[pl.ds(g * _L, _L)]
                a = plsc.load_gather(dinv_v, [s16])
                b = plsc.load_gather(dinv_v, [d16])
                obuf[pl.ds(g * _L, _L)] = a * w16 * b

            pltpu.sync_copy(obuf, out_hbm.at[pl.ds(base, _CB)])
            return carry

        lax.fori_loop(0, ncb, chunk, 0)

    return norm_kernel(src, dst, ew, dinv)


def _sc_hop(h, src, dst, norm):
    n, f = h.shape
    e = src.shape[0]
    epw = e // _NW
    nch = epw // _CH
    rpt = n // _NS
    zr = rpt // 5
    nsl = f // _L
    mesh = plsc.VectorSubcoreMesh(core_axis_name="c", subcore_axis_name="s")

    @functools.partial(
        pl.kernel,
        out_type=jax.ShapeDtypeStruct((_NC, n, f), jnp.float32),
        mesh=mesh,
        scratch_types=[
            pltpu.VMEM((_CH,), jnp.int32),
            pltpu.VMEM((_CH,), jnp.int32),
            pltpu.VMEM((_CH,), jnp.float32),
            pltpu.VMEM((_CH, f), jnp.float32),
            pltpu.VMEM((zr, f), jnp.float32),
            pltpu.VMEM_SHARED((n, f), jnp.float32),
            pltpu.SemaphoreType.DMA,
        ],
    )
    def hop_kernel(h_hbm, src_hbm, dst_hbm, norm_hbm, out_hbm,
                   sidx, didx, nbuf, rows, zbuf, acc, sem):
        cid = lax.axis_index("c")
        tid = lax.axis_index("s")
        wid = tid * _NC + cid
        zero16 = jnp.zeros((_L,), jnp.float32)

        @functools.partial(plsc.parallel_loop, 0, zr)
        def _zb(r):
            for k in range(nsl):
                zbuf[r, pl.ds(k * _L, _L)] = zero16

        for r in range(rpt // zr):
            pltpu.sync_copy(zbuf, acc.at[pl.ds(tid * rpt + r * zr, zr)])
        plsc.subcore_barrier()

        def chunk(c, carry):
            base = wid * epw + c * _CH
            pltpu.sync_copy(src_hbm.at[pl.ds(base, _CH)], sidx)
            pltpu.sync_copy(dst_hbm.at[pl.ds(base, _CH)], didx)
            pltpu.sync_copy(norm_hbm.at[pl.ds(base, _CH)], nbuf)
            pltpu.async_copy(h_hbm.at[sidx], rows, sem).wait()

            @functools.partial(plsc.parallel_loop, 0, _CH)
            def _scale(i):
                nb = plsc.load_gather(nbuf, [jnp.full((_L,), i, jnp.int32)])
                for k in range(nsl):
                    rows[i, pl.ds(k * _L, _L)] = rows[i, pl.ds(k * _L, _L)] * nb

            pltpu.sync_copy(rows, acc.at[didx], add=True)
            return carry

        lax.fori_loop(0, nch, chunk, 0)
        plsc.subcore_barrier()
        pltpu.sync_copy(acc.at[pl.ds(tid * rpt, rpt)],
                        out_hbm.at[cid, pl.ds(tid * rpt, rpt)])

    return hop_kernel(h, src, dst, norm)


_BN = 2000


def _tc_dinv(degp):
    nw, n = degp.shape

    def body(p_ref, o_ref):
        deg = jnp.sum(p_ref[...], axis=0)
        o_ref[...] = jnp.where(deg > 0, lax.rsqrt(deg), 0.0)

    return pl.pallas_call(
        body,
        out_shape=jax.ShapeDtypeStruct((n,), jnp.float32),
    )(degp)


def _tc_pairsum(p):
    _, n, f = p.shape
    grid = (n // _BN,)

    def body(p_ref, o_ref):
        o_ref[...] = p_ref[0] + p_ref[1]

    return pl.pallas_call(
        body,
        grid=grid,
        in_specs=[pl.BlockSpec((_NC, _BN, f), lambda i: (0, i, 0))],
        out_specs=pl.BlockSpec((_BN, f), lambda i: (i, 0)),
        out_shape=jax.ShapeDtypeStruct((n, f), jnp.float32),
    )(p)


def _tc_mm4(h0, h1, h2, p3, w, b):
    n, f = h0.shape
    hh = w.shape[1]
    grid = (n // _BN,)

    def body(h0_ref, h1_ref, h2_ref, p_ref, w_ref, b_ref, y_ref, s_ref, acc):
        i = pl.program_id(0)
        h3 = p_ref[0] + p_ref[1]
        z = jnp.dot(h0_ref[...], w_ref[0:f, :], preferred_element_type=jnp.float32)
        z += jnp.dot(h1_ref[...], w_ref[f:2 * f, :], preferred_element_type=jnp.float32)
        z += jnp.dot(h2_ref[...], w_ref[2 * f:3 * f, :], preferred_element_type=jnp.float32)
        z += jnp.dot(h3, w_ref[3 * f:4 * f, :], preferred_element_type=jnp.float32)
        y = jnp.maximum(z + b_ref[...], 0.0)
        y_ref[...] = y

        @pl.when(i == 0)
        def _():
            acc[...] = jnp.zeros_like(acc)

        acc[0:1, :] += jnp.sum(y, axis=0, keepdims=True)
        acc[1:2, :] += jnp.sum(y * y, axis=0, keepdims=True)
        s_ref[...] = acc[0:2, :]

    return pl.pallas_call(
        body,
        grid=grid,
        in_specs=[
            pl.BlockSpec((_BN, f), lambda i: (i, 0)),
            pl.BlockSpec((_BN, f), lambda i: (i, 0)),
            pl.BlockSpec((_BN, f), lambda i: (i, 0)),
            pl.BlockSpec((_NC, _BN, f), lambda i: (0, i, 0)),
            pl.BlockSpec((4 * f, hh), lambda i: (0, 0)),
            pl.BlockSpec((1, hh), lambda i: (0, 0)),
        ],
        out_specs=[
            pl.BlockSpec((_BN, hh), lambda i: (i, 0)),
            pl.BlockSpec((2, hh), lambda i: (0, 0)),
        ],
        out_shape=[
            jax.ShapeDtypeStruct((n, hh), jnp.float32),
            jax.ShapeDtypeStruct((2, hh), jnp.float32),
        ],
        scratch_shapes=[pltpu.VMEM((8, hh), jnp.float32)],
    )(h0, h1, h2, p3, w, b.reshape(1, hh))


def _tc_mm2(x1, x2, w, b):
    n, f = x1.shape
    hh = w.shape[1]
    grid = (n // _BN,)

    def body(x1_ref, x2_ref, w_ref, b_ref, y_ref, s_ref, acc):
        i = pl.program_id(0)
        z = jnp.dot(x1_ref[...], w_ref[0:f, :], preferred_element_type=jnp.float32)
        z += jnp.dot(x2_ref[...], w_ref[f:2 * f, :], preferred_element_type=jnp.float32)
        y = jnp.maximum(z + b_ref[...], 0.0)
        y_ref[...] = y

        @pl.when(i == 0)
        def _():
            acc[...] = jnp.zeros_like(acc)

        acc[0:1, :] += jnp.sum(y, axis=0, keepdims=True)
        acc[1:2, :] += jnp.sum(y * y, axis=0, keepdims=True)
        s_ref[...] = acc[0:2, :]

    return pl.pallas_call(
        body,
        grid=grid,
        in_specs=[
            pl.BlockSpec((_BN, f), lambda i: (i, 0)),
            pl.BlockSpec((_BN, f), lambda i: (i, 0)),
            pl.BlockSpec((2 * f, hh), lambda i: (0, 0)),
            pl.BlockSpec((1, hh), lambda i: (0, 0)),
        ],
        out_specs=[
            pl.BlockSpec((_BN, hh), lambda i: (i, 0)),
            pl.BlockSpec((2, hh), lambda i: (0, 0)),
        ],
        out_shape=[
            jax.ShapeDtypeStruct((n, hh), jnp.float32),
            jax.ShapeDtypeStruct((2, hh), jnp.float32),
        ],
        scratch_shapes=[pltpu.VMEM((8, hh), jnp.float32)],
    )(x1, x2, w, b.reshape(1, hh))


def _tc_bnorm(y, s, g, be):
    n, hh = y.shape
    grid = (n // _BN,)

    def body(y_ref, s_ref, g_ref, be_ref, o_ref):
        mu = s_ref[0:1, :] / n
        var = s_ref[1:2, :] / n - mu * mu
        inv = lax.rsqrt(var + 1e-5)
        o_ref[...] = (y_ref[...] - mu) * inv * g_ref[...] + be_ref[...]

    return pl.pallas_call(
        body,
        grid=grid,
        in_specs=[
            pl.BlockSpec((_BN, hh), lambda i: (i, 0)),
            pl.BlockSpec((2, hh), lambda i: (0, 0)),
            pl.BlockSpec((1, hh), lambda i: (0, 0)),
            pl.BlockSpec((1, hh), lambda i: (0, 0)),
        ],
        out_specs=pl.BlockSpec((_BN, hh), lambda i: (i, 0)),
        out_shape=jax.ShapeDtypeStruct((n, hh), jnp.float32),
    )(y, s, g.reshape(1, hh), be.reshape(1, hh))


def _tc_pool(ym, s, g, be, batch3d, nseg):
    n, hh = ym.shape
    grid = (n // _BN,)
    neg = jnp.float32(-jnp.inf)

    def body(y_ref, s_ref, g_ref, be_ref, b_ref, o_ref, acc):
        i = pl.program_id(0)
        mu = s_ref[0:1, :] / n
        var = s_ref[1:2, :] / n - mu * mu
        inv = lax.rsqrt(var + 1e-5)
        u = (y_ref[...] - mu) * inv * g_ref[...] + be_ref[...]
        b = b_ref[0, 0, :]

        @pl.when(i == 0)
        def _():
            acc[...] = jnp.full_like(acc, neg)

        for seg in range(nseg):
            m = jnp.max(jnp.where((b == seg)[:, None], u, neg), axis=0)
            acc[seg:seg + 1, :] = jnp.maximum(acc[seg:seg + 1, :], m[None, :])
        o_ref[...] = acc[...]

    return pl.pallas_call(
        body,
        grid=grid,
        in_specs=[
            pl.BlockSpec((_BN, hh), lambda i: (i, 0)),
            pl.BlockSpec((2, hh), lambda i: (0, 0)),
            pl.BlockSpec((1, hh), lambda i: (0, 0)),
            pl.BlockSpec((1, hh), lambda i: (0, 0)),
            pl.BlockSpec((1, 1, _BN), lambda i: (i, 0, 0)),
        ],
        out_specs=pl.BlockSpec((nseg, hh), lambda i: (0, 0)),
        out_shape=jax.ShapeDtypeStruct((nseg, hh), jnp.float32),
        scratch_shapes=[pltpu.VMEM((nseg, hh), jnp.float32)],
    )(ym, s, g.reshape(1, hh), be.reshape(1, hh), batch3d)


def _tc_head(pooled, wf1, bf1, gf1, bef1, wf2, bf2, gf2, bef2, wf3, bf3):
    g, hh = pooled.shape
    h2 = wf2.shape[1]
    c = wf3.shape[1]

    def bn(t, gg, bb):
        mu = jnp.mean(t, axis=0, keepdims=True)
        var = jnp.mean((t - mu) * (t - mu), axis=0, keepdims=True)
        return (t - mu) * lax.rsqrt(var + 1e-5) * gg + bb

    def body(p_ref, w1_ref, b1_ref, g1_ref, e1_ref, w2_ref, b2_ref, g2_ref,
             e2_ref, w3_ref, b3_ref, o_ref):
        t = jnp.maximum(jnp.dot(p_ref[...], w1_ref[...],
                                preferred_element_type=jnp.float32)
                        + b1_ref[...], 0.0)
        t = bn(t, g1_ref[...], e1_ref[...])
        t = jnp.maximum(jnp.dot(t, w2_ref[...],
                                preferred_element_type=jnp.float32)
                        + b2_ref[...], 0.0)
        t = bn(t, g2_ref[...], e2_ref[...])
        o = jnp.dot(t, w3_ref[...], preferred_element_type=jnp.float32) + b3_ref[...]
        m = jnp.max(o, axis=1, keepdims=True)
        lse = m + jnp.log(jnp.sum(jnp.exp(o - m), axis=1, keepdims=True))
        o_ref[...] = o - lse

    return pl.pallas_call(
        body,
        out_shape=jax.ShapeDtypeStruct((g, c), jnp.float32),
    )(pooled, wf1, bf1.reshape(1, hh), gf1.reshape(1, hh), bef1.reshape(1, hh),
      wf2, bf2.reshape(1, h2), gf2.reshape(1, h2), bef2.reshape(1, h2),
      wf3, bf3.reshape(1, c))


def kernel(x, edge_index, batch, edge_attr, W1, b1, g1, be1, W2, b2, g2, be2,
           Wm, bm, gm, bem, Wf1, bf1, gf1, bef1, Wf2, bf2, gf2, bef2, Wf3, bf3):
    n, f = x.shape
    src = edge_index[0]
    dst = edge_index[1]

    degp = _sc_degree(dst, edge_attr, n)
    dinv = _tc_dinv(degp)
    norm = _sc_norm(src, dst, edge_attr, dinv)

    p1 = _sc_hop(x, src, dst, norm)
    h1 = _tc_pairsum(p1)
    p2 = _sc_hop(h1, src, dst, norm)
    h2 = _tc_pairsum(p2)
    p3 = _sc_hop(h2, src, dst, norm)
    y1, s1 = _tc_mm4(x, h1, h2, p3, W1, b1)
    x1 = _tc_bnorm(y1, s1, g1, be1)

    q1 = _sc_hop(x1, src, dst, norm)
    k1 = _tc_pairsum(q1)
    q2 = _sc_hop(k1, src, dst, norm)
    k2 = _tc_pairsum(q2)
    q3 = _sc_hop(k2, src, dst, norm)
    y2, s2 = _tc_mm4(x1, k1, k2, q3, W2, b2)
    x2 = _tc_bnorm(y2, s2, g2, be2)

    ym, sm = _tc_mm2(x1, x2, Wm, bm)
    nseg = 64
    batch3d = batch.reshape(n // _BN, 1, _BN)
    pooled = _tc_pool(ym, sm, gm, bem, batch3d, nseg)

    return _tc_head(pooled, Wf1, bf1, gf1, bef1, Wf2, bf2, gf2, bef2, Wf3, bf3)


# speedup vs baseline: 5.5975x; 5.5975x over previous
import functools

import jax
import jax.numpy as jnp
from jax import lax
from jax.experimental import pallas as pl
from jax.experimental.pallas import tpu as pltpu
from jax.experimental.pallas import tpu_sc as plsc

_NC = 2
_NS = 16
_NW = _NC * _NS
_L = 16

_CH = 80
_CB = 2000


def _sc_degree(dst, ew, n):
    e = dst.shape[0]
    epw = e // _NW
    ncb = epw // _CB
    mesh = plsc.VectorSubcoreMesh(core_axis_name="c", subcore_axis_name="s", num_cores=_NC, num_subcores=_NS)

    @functools.partial(
        pl.kernel,
        out_type=jax.ShapeDtypeStruct((_NW, n), jnp.float32),
        mesh=mesh,
        compiler_params=pltpu.CompilerParams(needs_layout_passes=False),
        scratch_types=[
            pltpu.VMEM((_CB,), jnp.int32),
            pltpu.VMEM((_CB,), jnp.float32),
            pltpu.VMEM((n,), jnp.float32),
        ],
    )
    def deg_kernel(dst_hbm, ew_hbm, out_hbm, dbuf, wbuf, acc):
        wid = lax.axis_index("s") * _NC + lax.axis_index("c")
        zero16 = jnp.zeros((_L,), jnp.float32)

        @functools.partial(plsc.parallel_loop, 0, n // _L)
        def _zero(i):
            acc[pl.ds(i * _L, _L)] = zero16

        def chunk(c, carry):
            base = wid * epw + c * _CB
            pltpu.sync_copy(dst_hbm.at[pl.ds(base, _CB)], dbuf)
            pltpu.sync_copy(ew_hbm.at[pl.ds(base, _CB)], wbuf)

            def group(g, carry2):
                d16 = dbuf[pl.ds(g * _L, _L)]
                w16 = wbuf[pl.ds(g * _L, _L)]
                plsc.addupdate_scatter(acc, [d16], w16)
                return carry2

            return lax.fori_loop(0, _CB // _L, group, carry)

        lax.fori_loop(0, ncb, chunk, 0)
        pltpu.sync_copy(acc, out_hbm.at[wid])

    return deg_kernel(dst, ew)


def _sc_norm(src, dst, ew, dinv):
    e = src.shape[0]
    n = dinv.shape[0]
    epw = e // _NW
    ncb = epw // _CB
    mesh = plsc.VectorSubcoreMesh(core_axis_name="c", subcore_axis_name="s", num_cores=_NC, num_subcores=_NS)

    @functools.partial(
        pl.kernel,
        out_type=jax.ShapeDtypeStruct((e,), jnp.float32),
        mesh=mesh,
        compiler_params=pltpu.CompilerParams(needs_layout_passes=False),
        scratch_types=[
            pltpu.VMEM((n,), jnp.float32),
            pltpu.VMEM((_CB,), jnp.int32),
            pltpu.VMEM((_CB,), jnp.int32),
            pltpu.VMEM((_CB,), jnp.float32),
            pltpu.VMEM((_CB,), jnp.float32),
        ],
    )
    def norm_kernel(src_hbm, dst_hbm, ew_hbm, dinv_hbm, out_hbm,
                    dinv_v, sbuf, dbuf, wbuf, obuf):
        wid = lax.axis_index("s") * _NC + lax.axis_index("c")
        pltpu.sync_copy(dinv_hbm, dinv_v)

        def chunk(c, carry):
            base = wid * epw + c * _CB
            pltpu.sync_copy(src_hbm.at[pl.ds(base, _CB)], sbuf)
            pltpu.sync_copy(dst_hbm.at[pl.ds(base, _CB)], dbuf)
            pltpu.sync_copy(ew_hbm.at[pl.ds(base, _CB)], wbuf)

            @functools.partial(plsc.parallel_loop, 0, _CB // _L)
            def _group(g):
                s16 = sbuf[pl.ds(g * _L, _L)]
                d16 = dbuf[pl.ds(g * _L, _L)]
                w16 = wbuf[pl.ds(g * _L, _L)]
                a = plsc.load_gather(dinv_v, [s16])
                b = plsc.load_gather(dinv_v, [d16])
                obuf[pl.ds(g * _L, _L)] = a * w16 * b

            pltpu.sync_copy(obuf, out_hbm.at[pl.ds(base, _CB)])
            return carry

        lax.fori_loop(0, ncb, chunk, 0)

    return norm_kernel(src, dst, ew, dinv)


def _sc_hop(h, src, dst, norm):
    n, f = h.shape
    e = src.shape[0]
    epw = e // _NW
    nch = epw // _CH
    zr = 200
    ncz = n // zr
    npt = -(-ncz // _NS)
    nsl = f // _L
    mesh = plsc.VectorSubcoreMesh(core_axis_name="c", subcore_axis_name="s", num_cores=_NC, num_subcores=_NS)

    @functools.partial(
        pl.kernel,
        out_type=jax.ShapeDtypeStruct((_NC, n, f), jnp.float32),
        mesh=mesh,
        compiler_params=pltpu.CompilerParams(needs_layout_passes=False),
        scratch_types=[
            pltpu.VMEM((_CH,), jnp.int32),
            pltpu.VMEM((_CH,), jnp.int32),
            pltpu.VMEM((_CH,), jnp.float32),
            pltpu.VMEM((_CH, f), jnp.float32),
            pltpu.VMEM((zr, f), jnp.float32),
            pltpu.VMEM_SHARED((n, f), jnp.float32),
            pltpu.SemaphoreType.DMA,
        ],
    )
    def hop_kernel(h_hbm, src_hbm, dst_hbm, norm_hbm, out_hbm,
                   sidx, didx, nbuf, rows, zbuf, acc, sem):
        cid = lax.axis_index("c")
        tid = lax.axis_index("s")
        wid = tid * _NC + cid
        zero16 = jnp.zeros((_L,), jnp.float32)

        @functools.partial(plsc.parallel_loop, 0, zr)
        def _zb(r):
            for k in range(nsl):
                zbuf[r, pl.ds(k * _L, _L)] = zero16

        for j in range(npt):
            ci = tid + j * _NS

            @pl.when(ci < ncz)
            def _():
                pltpu.sync_copy(zbuf, acc.at[pl.ds(ci * zr, zr)])
        plsc.subcore_barrier()

        def chunk(c, carry):
            base = wid * epw + c * _CH
            pltpu.sync_copy(src_hbm.at[pl.ds(base, _CH)], sidx)
            pltpu.sync_copy(dst_hbm.at[pl.ds(base, _CH)], didx)
            pltpu.sync_copy(norm_hbm.at[pl.ds(base, _CH)], nbuf)
            pltpu.async_copy(h_hbm.at[sidx], rows, sem).wait()

            @functools.partial(plsc.parallel_loop, 0, _CH)
            def _scale(i):
                nb = plsc.load_gather(nbuf, [jnp.full((_L,), i, jnp.int32)])
                for k in range(nsl):
                    rows[i, pl.ds(k * _L, _L)] = rows[i, pl.ds(k * _L, _L)] * nb

            pltpu.sync_copy(rows, acc.at[didx], add=True)
            return carry

        lax.fori_loop(0, nch, chunk, 0)
        plsc.subcore_barrier()
        for j in range(npt):
            ci = tid + j * _NS

            @pl.when(ci < ncz)
            def _():
                pltpu.sync_copy(acc.at[pl.ds(ci * zr, zr)],
                                out_hbm.at[cid, pl.ds(ci * zr, zr)])

    return hop_kernel(h, src, dst, norm)


_BN = 2000


def _tc_dinv(degp):
    nw, n = degp.shape

    def body(p_ref, o_ref):
        deg = jnp.sum(p_ref[...], axis=0)
        o_ref[...] = jnp.where(deg > 0, lax.rsqrt(deg), 0.0)

    return pl.pallas_call(
        body,
        out_shape=jax.ShapeDtypeStruct((n,), jnp.float32),
    )(degp)


def _tc_pairsum(p):
    _, n, f = p.shape
    grid = (n // _BN,)

    def body(p_ref, o_ref):
        o_ref[...] = p_ref[0] + p_ref[1]

    return pl.pallas_call(
        body,
        grid=grid,
        in_specs=[pl.BlockSpec((_NC, _BN, f), lambda i: (0, i, 0))],
        out_specs=pl.BlockSpec((_BN, f), lambda i: (i, 0)),
        out_shape=jax.ShapeDtypeStruct((n, f), jnp.float32),
    )(p)


def _tc_mm4(h0, h1, h2, p3, w, b):
    n, f = h0.shape
    hh = w.shape[1]
    grid = (n // _BN,)

    def body(h0_ref, h1_ref, h2_ref, p_ref, w_ref, b_ref, y_ref, s_ref, acc):
        i = pl.program_id(0)
        h3 = p_ref[0] + p_ref[1]
        z = jnp.dot(h0_ref[...], w_ref[0:f, :], preferred_element_type=jnp.float32)
        z += jnp.dot(h1_ref[...], w_ref[f:2 * f, :], preferred_element_type=jnp.float32)
        z += jnp.dot(h2_ref[...], w_ref[2 * f:3 * f, :], preferred_element_type=jnp.float32)
        z += jnp.dot(h3, w_ref[3 * f:4 * f, :], preferred_element_type=jnp.float32)
        y = jnp.maximum(z + b_ref[...], 0.0)
        y_ref[...] = y

        @pl.when(i == 0)
        def _():
            acc[...] = jnp.zeros_like(acc)

        acc[0:1, :] += jnp.sum(y, axis=0, keepdims=True)
        acc[1:2, :] += jnp.sum(y * y, axis=0, keepdims=True)
        s_ref[...] = acc[0:2, :]

    return pl.pallas_call(
        body,
        grid=grid,
        in_specs=[
            pl.BlockSpec((_BN, f), lambda i: (i, 0)),
            pl.BlockSpec((_BN, f), lambda i: (i, 0)),
            pl.BlockSpec((_BN, f), lambda i: (i, 0)),
            pl.BlockSpec((_NC, _BN, f), lambda i: (0, i, 0)),
            pl.BlockSpec((4 * f, hh), lambda i: (0, 0)),
            pl.BlockSpec((1, hh), lambda i: (0, 0)),
        ],
        out_specs=[
            pl.BlockSpec((_BN, hh), lambda i: (i, 0)),
            pl.BlockSpec((2, hh), lambda i: (0, 0)),
        ],
        out_shape=[
            jax.ShapeDtypeStruct((n, hh), jnp.float32),
            jax.ShapeDtypeStruct((2, hh), jnp.float32),
        ],
        scratch_shapes=[pltpu.VMEM((8, hh), jnp.float32)],
    )(h0, h1, h2, p3, w, b.reshape(1, hh))


def _tc_mm2(x1, x2, w, b):
    n, f = x1.shape
    hh = w.shape[1]
    grid = (n // _BN,)

    def body(x1_ref, x2_ref, w_ref, b_ref, y_ref, s_ref, acc):
        i = pl.program_id(0)
        z = jnp.dot(x1_ref[...], w_ref[0:f, :], preferred_element_type=jnp.float32)
        z += jnp.dot(x2_ref[...], w_ref[f:2 * f, :], preferred_element_type=jnp.float32)
        y = jnp.maximum(z + b_ref[...], 0.0)
        y_ref[...] = y

        @pl.when(i == 0)
        def _():
            acc[...] = jnp.zeros_like(acc)

        acc[0:1, :] += jnp.sum(y, axis=0, keepdims=True)
        acc[1:2, :] += jnp.sum(y * y, axis=0, keepdims=True)
        s_ref[...] = acc[0:2, :]

    return pl.pallas_call(
        body,
        grid=grid,
        in_specs=[
            pl.BlockSpec((_BN, f), lambda i: (i, 0)),
            pl.BlockSpec((_BN, f), lambda i: (i, 0)),
            pl.BlockSpec((2 * f, hh), lambda i: (0, 0)),
            pl.BlockSpec((1, hh), lambda i: (0, 0)),
        ],
        out_specs=[
            pl.BlockSpec((_BN, hh), lambda i: (i, 0)),
            pl.BlockSpec((2, hh), lambda i: (0, 0)),
        ],
        out_shape=[
            jax.ShapeDtypeStruct((n, hh), jnp.float32),
            jax.ShapeDtypeStruct((2, hh), jnp.float32),
        ],
        scratch_shapes=[pltpu.VMEM((8, hh), jnp.float32)],
    )(x1, x2, w, b.reshape(1, hh))


def _tc_bnorm(y, s, g, be):
    n, hh = y.shape
    grid = (n // _BN,)

    def body(y_ref, s_ref, g_ref, be_ref, o_ref):
        mu = s_ref[0:1, :] / n
        var = s_ref[1:2, :] / n - mu * mu
        inv = lax.rsqrt(var + 1e-5)
        o_ref[...] = (y_ref[...] - mu) * inv * g_ref[...] + be_ref[...]

    return pl.pallas_call(
        body,
        grid=grid,
        in_specs=[
            pl.BlockSpec((_BN, hh), lambda i: (i, 0)),
            pl.BlockSpec((2, hh), lambda i: (0, 0)),
            pl.BlockSpec((1, hh), lambda i: (0, 0)),
            pl.BlockSpec((1, hh), lambda i: (0, 0)),
        ],
        out_specs=pl.BlockSpec((_BN, hh), lambda i: (i, 0)),
        out_shape=jax.ShapeDtypeStruct((n, hh), jnp.float32),
    )(y, s, g.reshape(1, hh), be.reshape(1, hh))


def _tc_pool(ym, s, g, be, batch3d, nseg):
    n, hh = ym.shape
    grid = (n // _BN,)
    neg = float("-inf")

    def body(y_ref, s_ref, g_ref, be_ref, b_ref, o_ref, acc):
        i = pl.program_id(0)
        mu = s_ref[0:1, :] / n
        var = s_ref[1:2, :] / n - mu * mu
        inv = lax.rsqrt(var + 1e-5)
        u = (y_ref[...] - mu) * inv * g_ref[...] + be_ref[...]
        b = b_ref[0]

        @pl.when(i == 0)
        def _():
            acc[...] = jnp.full_like(acc, neg)

        for seg in range(nseg):
            m = jnp.max(jnp.where(b == seg, u, neg), axis=0)
            acc[seg:seg + 1, :] = jnp.maximum(acc[seg:seg + 1, :], m[None, :])
        o_ref[...] = acc[...]

    return pl.pallas_call(
        body,
        grid=grid,
        in_specs=[
            pl.BlockSpec((_BN, hh), lambda i: (i, 0)),
            pl.BlockSpec((2, hh), lambda i: (0, 0)),
            pl.BlockSpec((1, hh), lambda i: (0, 0)),
            pl.BlockSpec((1, hh), lambda i: (0, 0)),
            pl.BlockSpec((1, _BN, 1), lambda i: (i, 0, 0)),
        ],
        out_specs=pl.BlockSpec((nseg, hh), lambda i: (0, 0)),
        out_shape=jax.ShapeDtypeStruct((nseg, hh), jnp.float32),
        scratch_shapes=[pltpu.VMEM((nseg, hh), jnp.float32)],
    )(ym, s, g.reshape(1, hh), be.reshape(1, hh), batch3d)


def _tc_head(pooled, wf1, bf1, gf1, bef1, wf2, bf2, gf2, bef2, wf3, bf3):
    g, hh = pooled.shape
    h2 = wf2.shape[1]
    c = wf3.shape[1]

    def bn(t, gg, bb):
        mu = jnp.mean(t, axis=0, keepdims=True)
        var = jnp.mean((t - mu) * (t - mu), axis=0, keepdims=True)
        return (t - mu) * lax.rsqrt(var + 1e-5) * gg + bb

    def body(p_ref, w1_ref, b1_ref, g1_ref, e1_ref, w2_ref, b2_ref, g2_ref,
             e2_ref, w3_ref, b3_ref, o_ref):
        t = jnp.maximum(jnp.dot(p_ref[...], w1_ref[...],
                                preferred_element_type=jnp.float32)
                        + b1_ref[...], 0.0)
        t = bn(t, g1_ref[...], e1_ref[...])
        t = jnp.maximum(jnp.dot(t, w2_ref[...],
                                preferred_element_type=jnp.float32)
                        + b2_ref[...], 0.0)
        t = bn(t, g2_ref[...], e2_ref[...])
        o = jnp.dot(t, w3_ref[...], preferred_element_type=jnp.float32) + b3_ref[...]
        m = jnp.max(o, axis=1, keepdims=True)
        lse = m + jnp.log(jnp.sum(jnp.exp(o - m), axis=1, keepdims=True))
        o_ref[...] = o - lse

    return pl.pallas_call(
        body,
        out_shape=jax.ShapeDtypeStruct((g, c), jnp.float32),
    )(pooled, wf1, bf1.reshape(1, hh), gf1.reshape(1, hh), bef1.reshape(1, hh),
      wf2, bf2.reshape(1, h2), gf2.reshape(1, h2), bef2.reshape(1, h2),
      wf3, bf3.reshape(1, c))


def kernel(x, edge_index, batch, edge_attr, W1, b1, g1, be1, W2, b2, g2, be2,
           Wm, bm, gm, bem, Wf1, bf1, gf1, bef1, Wf2, bf2, gf2, bef2, Wf3, bf3):
    n, f = x.shape
    src = edge_index[0]
    dst = edge_index[1]

    degp = _sc_degree(dst, edge_attr, n)
    dinv = _tc_dinv(degp)
    norm = _sc_norm(src, dst, edge_attr, dinv)

    p1 = _sc_hop(x, src, dst, norm)
    h1 = _tc_pairsum(p1)
    p2 = _sc_hop(h1, src, dst, norm)
    h2 = _tc_pairsum(p2)
    p3 = _sc_hop(h2, src, dst, norm)
    y1, s1 = _tc_mm4(x, h1, h2, p3, W1, b1)
    x1 = _tc_bnorm(y1, s1, g1, be1)

    q1 = _sc_hop(x1, src, dst, norm)
    k1 = _tc_pairsum(q1)
    q2 = _sc_hop(k1, src, dst, norm)
    k2 = _tc_pairsum(q2)
    q3 = _sc_hop(k2, src, dst, norm)
    y2, s2 = _tc_mm4(x1, k1, k2, q3, W2, b2)
    x2 = _tc_bnorm(y2, s2, g2, be2)

    ym, sm = _tc_mm2(x1, x2, Wm, bm)
    nseg = 64
    batch3d = batch.reshape(n // _BN, _BN, 1)
    pooled = _tc_pool(ym, sm, gm, bem, batch3d, nseg)

    return _tc_head(pooled, Wf1, bf1, gf1, bef1, Wf2, bf2, gf2, bef2, Wf3, bf3)
